# baseline (speedup 1.0000x reference)
import functools

import jax
import jax.numpy as jnp
from jax import lax
from jax.experimental import pallas as pl
from jax.experimental.pallas import tpu as pltpu
from jax.experimental.pallas import tpu_sc as plsc

_N = 10000
_E = 320000
_D = 128
_DE = 16
_L = 5

_NC = 2
_NS = 16
_NW = _NC * _NS
_EPT = _E // _NW
_CH = 80
_NCHUNK = _EPT // _CH
_NP = 10240
_SPT = _NP // _NS
_LAST = _N - (_NS - 1) * _SPT

_mesh = plsc.VectorSubcoreMesh(core_axis_name="c", subcore_axis_name="s")

_f32 = jnp.float32
_i32 = jnp.int32


def _hi(a):
    return a.astype(jnp.bfloat16).astype(_f32)


def _hi_host(a):
    i = lax.bitcast_convert_type(a, _i32)
    r = (i + 0x7FFF + ((i >> 16) & 1)) & jnp.int32(-65536)
    return lax.bitcast_convert_type(r, _f32)


def _hi_vec(x):
    i = plsc.bitcast(x, _i32)
    r = (i + 0x7FFF + ((i >> 16) & 1)) & jnp.int32(-65536)
    return plsc.bitcast(r, _f32)


def _dot(a, b):
    return jnp.dot(a, b, preferred_element_type=_f32,
                   precision=lax.Precision.HIGHEST)


@functools.partial(
    pl.kernel,
    out_type=(jax.ShapeDtypeStruct((_NC, _N, _DE), _f32),
              jax.ShapeDtypeStruct((_NC, _N, _DE), _f32)),
    mesh=_mesh,
    scratch_types=[
        pltpu.VMEM((_EPT,), _i32),
        pltpu.VMEM((_EPT,), _i32),
        pltpu.VMEM((_CH, _DE), _f32),
        pltpu.VMEM((_CH, _DE), _f32),
        pltpu.VMEM((_CH, _DE), _f32),
        pltpu.VMEM((_CH, _DE), _f32),
        pltpu.VMEM((_CH, _DE), _f32),
        pltpu.VMEM((_CH, _DE), _f32),
        pltpu.VMEM((_CH, _DE), _f32),
        pltpu.VMEM_SHARED((_NP, _DE), _f32),
        pltpu.VMEM_SHARED((_NP, _DE), _f32),
        pltpu.SemaphoreType.DMA,
        pltpu.SemaphoreType.DMA,
        pltpu.SemaphoreType.DMA,
        pltpu.SemaphoreType.DMA,
        pltpu.SemaphoreType.DMA,
        pltpu.SemaphoreType.DMA,
    ],
    compiler_params=pltpu.CompilerParams(use_tc_tiling_on_sc=False,
                                         needs_layout_passes=False),
)
def _sc_precompute(p_hbm, eah_hbm, src_hbm, dst_hbm, z16_hbm,
                   outsq_hbm, outea_hbm,
                   src_v, dst_v, ps_a, ps_b, pd_a, pd_b, ea_a, ea_b, stage,
                   accsq, accea,
                   sps_a, sps_b, spd_a, spd_b, sea_a, sea_b):
    c = lax.axis_index("c")
    s = lax.axis_index("s")
    w = c * _NS + s
    ebase = pl.multiple_of(w * _EPT, 8)
    pltpu.sync_copy(src_hbm.at[pl.ds(ebase, _EPT)], src_v)
    pltpu.sync_copy(dst_hbm.at[pl.ds(ebase, _EPT)], dst_v)
    pltpu.sync_copy(z16_hbm, accsq.at[pl.ds(s * _SPT, _SPT)])
    pltpu.sync_copy(z16_hbm, accea.at[pl.ds(s * _SPT, _SPT)])
    const_row = jnp.where(lax.iota(_i32, 16) == 1, 1.0, 0.0).astype(_f32)
    for r in range(_CH):
        stage[r, :] = const_row
    plsc.subcore_barrier()

    def fetch(i, psb, pss, pdb, pds, eab, eas):
        idx = src_v.at[pl.ds(i * _CH, _CH)]
        idxd = dst_v.at[pl.ds(i * _CH, _CH)]
        pltpu.async_copy(p_hbm.at[idx], psb, pss)
        pltpu.async_copy(p_hbm.at[idxd], pdb, pds)
        base = pl.multiple_of(ebase + i * _CH, 8)
        pltpu.async_copy(eah_hbm.at[pl.ds(base, _CH)], eab, eas)

    def wait(psb, pss, pdb, pds, eab, eas):
        pltpu.make_async_copy(p_hbm.at[src_v.at[pl.ds(0, _CH)]], psb, pss).wait()
        pltpu.make_async_copy(p_hbm.at[src_v.at[pl.ds(0, _CH)]], pdb, pds).wait()
        pltpu.make_async_copy(eah_hbm.at[pl.ds(0, _CH)], eab, eas).wait()

    def compute_sq(psb, pdb):
        for g in range(_CH // 16):
            rows = lax.iota(_i32, 16) + 16 * g
            col = lambda j: jnp.zeros((16,), _i32) + j
            ld = lambda buf, j: plsc.load_gather(buf, [rows, col(j)])
            dx = ld(psb, 0) - ld(pdb, 0)
            dy = ld(psb, 1) - ld(pdb, 1)
            dz = ld(psb, 2) - ld(pdb, 2)
            sq = (dx * dx + dy * dy) + dz * dz
            plsc.store_scatter(stage, [rows, col(0)], _hi_vec(sq))

    def scatter(i, eab):
        idxd = dst_v.at[pl.ds(i * _CH, _CH)]
        pltpu.sync_copy(stage, accsq.at[idxd], add=True)
        pltpu.sync_copy(eab, accea.at[idxd], add=True)

    fetch(0, ps_a, sps_a, pd_a, spd_a, ea_a, sea_a)

    def body(j, carry):
        i0 = j * 2
        fetch(i0 + 1, ps_b, sps_b, pd_b, spd_b, ea_b, sea_b)
        wait(ps_a, sps_a, pd_a, spd_a, ea_a, sea_a)
        compute_sq(ps_a, pd_a)
        scatter(i0, ea_a)
        fetch(i0 + 2, ps_a, sps_a, pd_a, spd_a, ea_a, sea_a)
        wait(ps_b, sps_b, pd_b, spd_b, ea_b, sea_b)
        compute_sq(ps_b, pd_b)
        scatter(i0 + 1, ea_b)
        return carry

    lax.fori_loop(0, (_NCHUNK - 1) // 2, body, 0)
    wait(ps_a, sps_a, pd_a, spd_a, ea_a, sea_a)
    compute_sq(ps_a, pd_a)
    scatter(_NCHUNK - 1, ea_a)
    plsc.subcore_barrier()

    @pl.when(s < _NS - 1)
    def _():
        pltpu.sync_copy(accsq.at[pl.ds(s * _SPT, _SPT)],
                        outsq_hbm.at[c, pl.ds(s * _SPT, _SPT)])
        pltpu.sync_copy(accea.at[pl.ds(s * _SPT, _SPT)],
                        outea_hbm.at[c, pl.ds(s * _SPT, _SPT)])

    @pl.when(s == _NS - 1)
    def _():
        pltpu.sync_copy(accsq.at[pl.ds((_NS - 1) * _SPT, _LAST)],
                        outsq_hbm.at[c, pl.ds((_NS - 1) * _SPT, _LAST)])
        pltpu.sync_copy(accea.at[pl.ds((_NS - 1) * _SPT, _LAST)],
                        outea_hbm.at[c, pl.ds((_NS - 1) * _SPT, _LAST)])


@functools.partial(
    pl.kernel,
    out_type=jax.ShapeDtypeStruct((_NC, _N, _D), _f32),
    mesh=_mesh,
    scratch_types=[
        pltpu.VMEM((_EPT,), _i32),
        pltpu.VMEM((_EPT,), _i32),
        pltpu.VMEM((_CH, _D), _f32),
        pltpu.VMEM((_CH, _D), _f32),
        pltpu.VMEM_SHARED((_NP, _D), _f32),
        pltpu.SemaphoreType.DMA,
        pltpu.SemaphoreType.DMA,
    ],
)
def _sc_spmm(t_hbm, src_hbm, dst_hbm, z128_hbm, out_hbm,
             src_v, dst_v, rows_a, rows_b, acc, sem_a, sem_b):
    c = lax.axis_index("c")
    s = lax.axis_index("s")
    w = c * _NS + s
    ebase = pl.multiple_of(w * _EPT, 8)
    pltpu.sync_copy(src_hbm.at[pl.ds(ebase, _EPT)], src_v)
    pltpu.sync_copy(dst_hbm.at[pl.ds(ebase, _EPT)], dst_v)
    pltpu.sync_copy(z128_hbm, acc.at[pl.ds(s * _SPT, _SPT)])
    plsc.subcore_barrier()

    def gather(i, buf, sem):
        return pltpu.async_copy(
            t_hbm.at[src_v.at[pl.ds(i * _CH, _CH)]], buf, sem)

    def wait(buf, sem):
        pltpu.make_async_copy(
            t_hbm.at[src_v.at[pl.ds(0, _CH)]], buf, sem).wait()

    def scatter(i, buf):
        pltpu.sync_copy(buf, acc.at[dst_v.at[pl.ds(i * _CH, _CH)]], add=True)

    gather(0, rows_a, sem_a)

    def body(j, carry):
        i0 = j * 2
        gather(i0 + 1, rows_b, sem_b)
        wait(rows_a, sem_a)
        scatter(i0, rows_a)
        gather(i0 + 2, rows_a, sem_a)
        wait(rows_b, sem_b)
        scatter(i0 + 1, rows_b)
        return carry

    lax.fori_loop(0, (_NCHUNK - 1) // 2, body, 0)
    wait(rows_a, sem_a)
    scatter(_NCHUNK - 1, rows_a)
    plsc.subcore_barrier()

    @pl.when(s < _NS - 1)
    def _():
        pltpu.sync_copy(acc.at[pl.ds(s * _SPT, _SPT)],
                        out_hbm.at[c, pl.ds(s * _SPT, _SPT)])

    @pl.when(s == _NS - 1)
    def _():
        pltpu.sync_copy(acc.at[pl.ds((_NS - 1) * _SPT, _LAST)],
                        out_hbm.at[c, pl.ds((_NS - 1) * _SPT, _LAST)])


_R = 1000
_G = _N // _R


def _tc_pre_body(x_ref, pos_ref, w_ref, b_ref, h_ref, t_ref, p_ref):
    h = jnp.maximum(_dot(_hi(x_ref[...]), w_ref[...]) + b_ref[...], 0.0)
    h_ref[...] = h
    t_ref[...] = _hi(h)
    pos = pos_ref[...]
    q = jnp.sum(pos * pos, axis=1, keepdims=True)
    one = jnp.ones_like(q)
    zer = jnp.zeros((pos.shape[0], 6), _f32)
    p_ref[...] = jnp.concatenate([pos, q, one, zer], axis=1)


def _tc_pre(x, pos_pad, w_in_h, b_in):
    return pl.pallas_call(
        _tc_pre_body,
        grid=(_G,),
        in_specs=[
            pl.BlockSpec((_R, _D), lambda i: (i, 0)),
            pl.BlockSpec((_R, 8), lambda i: (i, 0)),
            pl.BlockSpec((_D, _D), lambda i: (0, 0)),
            pl.BlockSpec((1, _D), lambda i: (0, 0)),
        ],
        out_specs=[
            pl.BlockSpec((_R, _D), lambda i: (i, 0)),
            pl.BlockSpec((_R, _D), lambda i: (i, 0)),
            pl.BlockSpec((_R, _DE), lambda i: (i, 0)),
        ],
        out_shape=[jax.ShapeDtypeStruct((_N, _D), _f32),
                   jax.ShapeDtypeStruct((_N, _D), _f32),
                   jax.ShapeDtypeStruct((_N, _DE), _f32)],
    )(x, pos_pad, w_in_h, b_in)


def _tc_layer_body(h_ref, sp_ref, asq_ref, aea_ref,
                   wa_ref, wb_ref, we_ref, wd_ref, bpre_ref,
                   wpost_ref, bpost_ref, out_ref, t_ref):
    h = h_ref[...]
    s = sp_ref[0] + sp_ref[1]
    asq = asq_ref[0] + asq_ref[1]
    eah = aea_ref[0] + aea_ref[1]
    ssh = asq[:, 0:1]
    deg = asq[:, 1:2]
    m = (_dot(s, wa_ref[...])
         + _dot(deg * _hi(h), wb_ref[...])
         + _dot(eah, we_ref[...])
         + ssh * wd_ref[...]
         + deg * bpre_ref[...])
    cat = jnp.concatenate([h, m], axis=1)
    hn = h + _dot(_hi(cat), wpost_ref[...]) + bpost_ref[...]
    out_ref[...] = hn
    t_ref[...] = _hi(hn)


def _tc_layer(h, sp, accsq, accea, wa, wb, we, wd, bpre, wpost, bpost):
    full = lambda r, c: pl.BlockSpec((r, c), lambda i: (0, 0))
    return pl.pallas_call(
        _tc_layer_body,
        grid=(_G,),
        in_specs=[
            pl.BlockSpec((_R, _D), lambda i: (i, 0)),
            pl.BlockSpec((_NC, _R, _D), lambda i: (0, i, 0)),
            pl.BlockSpec((_NC, _R, _DE), lambda i: (0, i, 0)),
            pl.BlockSpec((_NC, _R, _DE), lambda i: (0, i, 0)),
            full(_D, _D), full(_D, _D), full(_DE, _D), full(1, _D), full(1, _D),
            full(2 * _D, _D), full(1, _D),
        ],
        out_specs=[pl.BlockSpec((_R, _D), lambda i: (i, 0)),
                   pl.BlockSpec((_R, _D), lambda i: (i, 0))],
        out_shape=[jax.ShapeDtypeStruct((_N, _D), _f32),
                   jax.ShapeDtypeStruct((_N, _D), _f32)],
    )(h, sp, accsq, accea, wa, wb, we, wd, bpre, wpost, bpost)


def _tc_readout_body(h_ref, w1_ref, b1_ref, w2t_ref, b2_ref, out_ref):
    h = h_ref[...]
    mean = jnp.mean(h, axis=0, keepdims=True)
    mx = jnp.max(h, axis=0, keepdims=True)
    mm = jnp.concatenate([mean, mx], axis=1)
    z = jnp.maximum(_dot(_hi(mm), w1_ref[...]) + b1_ref[...], 0.0)
    out_ref[...] = jnp.sum(_hi(z) * w2t_ref[...], axis=1, keepdims=True) + b2_ref[...]


def _tc_readout(h, w1_h, b1, w2t_h, b2):
    return pl.pallas_call(
        _tc_readout_body,
        out_shape=jax.ShapeDtypeStruct((1, 1), _f32),
    )(h, w1_h, b1, w2t_h, b2)


def kernel(x, pos, edge_index, edge_attr, W_in, b_in, W_pre, b_pre,
           W_post, b_post, W_r1, b_r1, W_r2, b_r2):
    src = edge_index[0]
    dst = edge_index[1]
    pos_pad = jnp.pad(pos, ((0, 0), (0, 5)))
    z16 = jnp.zeros((_SPT, _DE), _f32)
    z128 = jnp.zeros((_SPT, _D), _f32)
    ea_h = _hi_host(edge_attr)

    h, t, p = _tc_pre(x, pos_pad, _hi_host(W_in), b_in.reshape(1, _D))
    accsq, accea = _sc_precompute(p, ea_h, src, dst, z16)

    for l in range(_L):
        sp = _sc_spmm(t, src, dst, z128)
        h, t = _tc_layer(
            h, sp, accsq, accea,
            _hi_host(W_pre[l, :_D]), _hi_host(W_pre[l, _D:2 * _D]),
            _hi_host(W_pre[l, 2 * _D:2 * _D + _DE]),
            _hi_host(W_pre[l, 2 * _D + _DE:]),
            b_pre[l].reshape(1, _D),
            _hi_host(W_post[l]), b_post[l].reshape(1, _D))

    return _tc_readout(h, _hi_host(W_r1), b_r1.reshape(1, _D),
                       _hi_host(W_r2.reshape(1, _D)), b_r2.reshape(1, 1))
